# Initial kernel scaffold; baseline (speedup 1.0000x reference)
import jax
import jax.numpy as jnp
from jax.experimental import pallas as pl


def kernel(img, fix_loc, fixation_size, sampling_grid, knn_idx):
    raise NotImplementedError("write your pallas kernel here")


# speedup vs baseline: 17.8532x; 17.8532x over previous
import functools

import jax
import jax.numpy as jnp
from jax import lax
from jax.experimental import pallas as pl
from jax.experimental.pallas import tpu as pltpu
from jax.experimental.pallas import tpu_sc as plsc

_NW = 32
_L = 16


def _round_half_even(x):
    y = x + 0.5
    t = y.astype(jnp.int32)
    tf = t.astype(jnp.float32)
    r = jnp.where(tf > y, t - 1, t)
    rf = r.astype(jnp.float32)
    tie = rf == y
    odd = (r & 1) == 1
    return jnp.where(tie & odd, r - 1, r)


def _build_sc(P, HW, H, W, N_out, K):
    J = N_out // _NW
    G = J * K
    assert N_out % _NW == 0 and J % _L == 0 and G % _L == 0

    mesh = plsc.VectorSubcoreMesh(core_axis_name="c", subcore_axis_name="s")

    @functools.partial(
        pl.kernel,
        mesh=mesh,
        out_type=jax.ShapeDtypeStruct((P * N_out,), jnp.float32),
        scratch_types=[
            pltpu.VMEM((G,), jnp.int32),
            pltpu.VMEM((G,), jnp.float32),
            pltpu.VMEM((G,), jnp.float32),
            pltpu.VMEM((G,), jnp.int32),
            pltpu.VMEM((G,), jnp.float32),
            pltpu.VMEM((G,), jnp.float32),
            pltpu.VMEM((G,), jnp.float32),
            pltpu.VMEM((J,), jnp.float32),
            pltpu.VMEM((3, _L), jnp.float32),
            pltpu.SemaphoreType.DMA,
            pltpu.SemaphoreType.DMA,
        ],
    )
    def sc_kernel(imgf, gridx, gridy, params, knnt, outf,
                  knn_v, gx_v, gy_v, fa_v, w_v, val_a, val_b, out_v, par_v,
                  sem0, sem1):
        cid = lax.axis_index("c")
        sid = lax.axis_index("s")
        wid = sid * 2 + cid

        pltpu.sync_copy(params, par_v)
        pltpu.sync_copy(knnt.at[wid], knn_v)
        pltpu.async_copy(gridx.at[knn_v], gx_v, sem0).wait()
        pltpu.async_copy(gridy.at[knn_v], gy_v, sem0).wait()

        fxv = par_v[0, :]
        fyv = par_v[1, :]
        sv = par_v[2, :]

        def addr_body(c, carry):
            sl16 = pl.ds(c * _L, _L)
            gx = gx_v[sl16]
            gy = gy_v[sl16]
            px = fxv + gx * sv
            py = fyv - gy * sv
            ix = _round_half_even(px)
            iy = _round_half_even(py)
            valid = (ix >= 0) & (ix <= W - 1) & (iy >= 0) & (iy <= H - 1)
            w16 = jnp.where(valid, jnp.float32(1.0 / K), jnp.float32(0.0))
            ixc = jnp.clip(ix, 0, W - 1)
            iyc = jnp.clip(iy, 0, H - 1)
            fa_v[sl16] = iyc * W + ixc
            w_v[sl16] = w16
            return carry

        lax.fori_loop(0, G // _L, addr_body, 0)

        bufs = (val_a, val_b)
        sems = (sem0, sem1)

        def issue(p, buf):
            return pltpu.async_copy(
                imgf.at[pl.ds(p * HW, HW)].at[fa_v], bufs[buf], sems[buf])

        cps = [issue(0, 0), None]
        for p in range(P):
            buf = p % 2
            if p + 1 < P:
                cps[1 - buf] = issue(p + 1, 1 - buf)
            cps[buf].wait()

            def acc_body(cc, carry, buf=buf):
                j0 = cc * _L
                a = bufs[buf][pl.ds(j0, _L)] * w_v[pl.ds(j0, _L)]
                for kk in range(1, K):
                    o = kk * J + j0
                    a = a + bufs[buf][pl.ds(o, _L)] * w_v[pl.ds(o, _L)]
                out_v[pl.ds(j0, _L)] = a
                return carry

            lax.fori_loop(0, J // _L, acc_body, 0)
            pltpu.sync_copy(out_v, outf.at[pl.ds(p * N_out + wid * J, J)])

    return sc_kernel


def kernel(img, fix_loc, fixation_size, sampling_grid, knn_idx):
    B, C, H, W = img.shape
    N_out, K = knn_idx.shape
    P = B * C
    J = N_out // _NW

    imgf = img.reshape(P * H * W)
    gridx = sampling_grid[:, 0]
    gridy = sampling_grid[:, 1]
    knnt = knn_idx.reshape(_NW, J, K).transpose(0, 2, 1).reshape(_NW, J * K)
    scale = jnp.float32(fixation_size) / jnp.float32(2.0)
    params = jnp.stack([
        jnp.broadcast_to(fix_loc[0].astype(jnp.float32), (_L,)),
        jnp.broadcast_to(fix_loc[1].astype(jnp.float32), (_L,)),
        jnp.broadcast_to(scale, (_L,)),
    ])

    outf = _build_sc(P, H * W, H, W, N_out, K)(imgf, gridx, gridy, params, knnt)
    return outf.reshape(B, C, N_out)
